# baseline (speedup 1.0000x reference)
import functools

import jax
import jax.numpy as jnp
from jax import lax
from jax.experimental import pallas as pl
from jax.experimental.pallas import tpu as pltpu
from jax.experimental.pallas import tpu_sc as plsc

B = 4096
CTX = 20
H = 64
NV = 100001

NC = 2
NS = 16
NW = NC * NS
B_PER_W = B // NW
CHUNKB = 32
NCHUNK = B_PER_W // CHUNKB
IDXC = CHUNKB * CTX
GSZ = 128
NG = IDXC // GSZ
NLANE = 16
HV = H // NLANE


def _sc_body(x_hbm, enc_hbm, ctx_hbm, idx_v, rows_v, out_v, sem):
    wid = lax.axis_index("s") * NC + lax.axis_index("c")
    base_row = wid * B_PER_W

    def chunk_body(c, carry):
        row0 = base_row + c * CHUNKB
        pltpu.sync_copy(x_hbm.at[pl.ds(row0 * CTX, IDXC)], idx_v)
        copies = []
        for g in range(NG):
            copies.append(
                pltpu.async_copy(
                    enc_hbm.at[idx_v.at[pl.ds(g * GSZ, GSZ)]],
                    rows_v.at[pl.ds(g * GSZ, GSZ)],
                    sem,
                )
            )
        for cp in copies:
            cp.wait()

        def brow(b, inner):
            r0 = b * CTX
            for h in range(HV):
                acc = rows_v[r0, pl.ds(h * NLANE, NLANE)]
                for j in range(1, CTX):
                    acc = acc + rows_v[r0 + j, pl.ds(h * NLANE, NLANE)]
                out_v[b, pl.ds(h * NLANE, NLANE)] = acc * (1.0 / CTX)
            return inner

        lax.fori_loop(0, CHUNKB, brow, 0)
        pltpu.sync_copy(out_v, ctx_hbm.at[pl.ds(row0, CHUNKB)])
        return carry

    lax.fori_loop(0, NCHUNK, chunk_body, 0)


_sc_gather_mean = functools.partial(
    pl.kernel,
    out_type=jax.ShapeDtypeStruct((B, H), jnp.float32),
    mesh=plsc.VectorSubcoreMesh(
        core_axis_name="c", subcore_axis_name="s", num_cores=NC, num_subcores=NS
    ),
    scratch_types=[
        pltpu.VMEM((IDXC,), jnp.int32),
        pltpu.VMEM((IDXC, H), jnp.float32),
        pltpu.VMEM((CHUNKB, H), jnp.float32),
        pltpu.SemaphoreType.DMA,
    ],
    compiler_params=pltpu.CompilerParams(use_tc_tiling_on_sc=False),
)(_sc_body)


MB = 512
NB = 2048


def _mm_body(ctx_ref, w_ref, o_ref):
    o_ref[...] = lax.dot_general(
        ctx_ref[...],
        w_ref[...],
        (((1,), (1,)), ((), ())),
        preferred_element_type=jnp.float32,
    )


def _tc_decode(context, W_dec):
    return pl.pallas_call(
        _mm_body,
        grid=(B // MB, pl.cdiv(NV, NB)),
        in_specs=[
            pl.BlockSpec((MB, H), lambda i, j: (i, 0)),
            pl.BlockSpec((NB, H), lambda i, j: (j, 0)),
        ],
        out_specs=pl.BlockSpec((MB, NB), lambda i, j: (i, j)),
        out_shape=jax.ShapeDtypeStruct((B, NV), jnp.float32),
    )(context, W_dec)


def kernel(x, W_enc, W_dec):
    x_flat = x.reshape(-1).astype(jnp.int32)
    context = _sc_gather_mean(x_flat, W_enc)
    return _tc_decode(context, W_dec)


# speedup vs baseline: 1.1051x; 1.1051x over previous
import functools

import jax
import jax.numpy as jnp
from jax import lax
from jax.experimental import pallas as pl
from jax.experimental.pallas import tpu as pltpu
from jax.experimental.pallas import tpu_sc as plsc

B = 4096
CTX = 20
H = 64
NV = 100001

NC = 2
NS = 16
NW = NC * NS
B_PER_W = B // NW
CHUNKB = 32
NCHUNK = B_PER_W // CHUNKB
IDXC = CHUNKB * CTX
GSZ = 128
NG = IDXC // GSZ
NLANE = 16
HV = H // NLANE


def _sc_body(x_hbm, enc_hbm, ctx_hbm, idx_v, rows_v, out_v, sem):
    wid = lax.axis_index("s") * NC + lax.axis_index("c")
    base_row = wid * B_PER_W

    def chunk_body(c, carry):
        row0 = base_row + c * CHUNKB
        pltpu.sync_copy(x_hbm.at[pl.ds(row0 * CTX, IDXC)], idx_v)
        copies = []
        for g in range(NG):
            copies.append(
                pltpu.async_copy(
                    enc_hbm.at[idx_v.at[pl.ds(g * GSZ, GSZ)]],
                    rows_v.at[pl.ds(g * GSZ, GSZ)],
                    sem,
                )
            )
        for cp in copies:
            cp.wait()

        def brow(b, inner):
            r0 = b * CTX
            for h in range(HV):
                acc = rows_v[r0, pl.ds(h * NLANE, NLANE)]
                for j in range(1, CTX):
                    acc = acc + rows_v[r0 + j, pl.ds(h * NLANE, NLANE)]
                out_v[b, pl.ds(h * NLANE, NLANE)] = acc * (1.0 / CTX)
            return inner

        lax.fori_loop(0, CHUNKB, brow, 0)
        pltpu.sync_copy(out_v, ctx_hbm.at[pl.ds(row0, CHUNKB)])
        return carry

    lax.fori_loop(0, NCHUNK, chunk_body, 0)


_sc_gather_mean = functools.partial(
    pl.kernel,
    out_type=jax.ShapeDtypeStruct((B, H), jnp.float32),
    mesh=plsc.VectorSubcoreMesh(
        core_axis_name="c", subcore_axis_name="s", num_cores=NC, num_subcores=NS
    ),
    scratch_types=[
        pltpu.VMEM((IDXC,), jnp.int32),
        pltpu.VMEM((IDXC, H), jnp.float32),
        pltpu.VMEM((CHUNKB, H), jnp.float32),
        pltpu.SemaphoreType.DMA,
    ],
    compiler_params=pltpu.CompilerParams(use_tc_tiling_on_sc=False),
)(_sc_body)


RING = 4
NSUB = 512
NB = RING * NSUB
NSTEP = pl.cdiv(NV, NB)
NFULL = NV // NSUB
TAIL_J = NFULL


def _dot_nt(a, b):
    return lax.dot_general(
        a, b, (((1,), (1,)), ((), ())), preferred_element_type=jnp.float32
    )


def _mm_body(ctx_ref, w_ref, o_hbm, acc, sems):
    j = pl.program_id(0)
    ctx = ctx_ref[...]
    for r in range(RING):
        @pl.when(j >= 1)
        def _():
            pltpu.make_async_copy(
                acc.at[r], o_hbm.at[:, pl.ds(0, NSUB)], sems.at[r]
            ).wait()

        acc[r] = _dot_nt(ctx, w_ref[pl.ds(r * NSUB, NSUB), :])
        col = j * NB + r * NSUB
        if r < RING - 1:
            pltpu.make_async_copy(
                acc.at[r], o_hbm.at[:, pl.ds(col, NSUB)], sems.at[r]
            ).start()
        else:
            @pl.when(j < NSTEP - 1)
            def _():
                pltpu.make_async_copy(
                    acc.at[r], o_hbm.at[:, pl.ds(col, NSUB)], sems.at[r]
                ).start()

    @pl.when(j == NSTEP - 1)
    def _():
        for r in range(RING - 1):
            pltpu.make_async_copy(
                acc.at[r], o_hbm.at[:, pl.ds(0, NSUB)], sems.at[r]
            ).wait()


def _tail_body(ctx_ref, w_ref, prev_ref, o_ref):
    del prev_ref
    o_ref[...] = _dot_nt(ctx_ref[...], w_ref[...])


def _tc_decode(context, W_dec):
    bulk = pl.pallas_call(
        _mm_body,
        grid=(NSTEP,),
        in_specs=[
            pl.BlockSpec((B, H), lambda j: (0, 0)),
            pl.BlockSpec((NB, H), lambda j: (j, 0)),
        ],
        out_specs=pl.BlockSpec(memory_space=pltpu.MemorySpace.HBM),
        out_shape=jax.ShapeDtypeStruct((B, NV), jnp.float32),
        scratch_shapes=[
            pltpu.VMEM((RING, B, NSUB), jnp.float32),
            pltpu.SemaphoreType.DMA((RING,)),
        ],
    )(context, W_dec)
    return pl.pallas_call(
        _tail_body,
        grid=(1,),
        in_specs=[
            pl.BlockSpec((B, H), lambda j: (0, 0)),
            pl.BlockSpec((NSUB, H), lambda j: (TAIL_J, 0)),
            pl.BlockSpec(memory_space=pltpu.MemorySpace.HBM),
        ],
        out_specs=pl.BlockSpec((B, NSUB), lambda j: (0, TAIL_J)),
        out_shape=jax.ShapeDtypeStruct((B, NV), jnp.float32),
        input_output_aliases={2: 0},
    )(context, W_dec, bulk)


def kernel(x, W_enc, W_dec):
    x_flat = x.reshape(-1).astype(jnp.int32)
    context = _sc_gather_mean(x_flat, W_enc)
    return _tc_decode(context, W_dec)
